# baseline (speedup 1.0000x reference)
import functools

import jax
import jax.numpy as jnp
from jax import lax
from jax.experimental import pallas as pl
from jax.experimental.pallas import tpu as pltpu
from jax.experimental.pallas import tpu_sc as plsc

NC = 2
NS = 16
NW = NC * NS
CHUNK = 128


def _round_up(a, m):
    return (a + m - 1) // m * m


def _make_deg(np_, rp, nchunk):
    mesh = plsc.VectorSubcoreMesh(core_axis_name="c", subcore_axis_name="s")

    @functools.partial(
        pl.kernel,
        out_type=jax.ShapeDtypeStruct((NC, np_, 16), jnp.float32),
        mesh=mesh,
        compiler_params=pltpu.CompilerParams(use_tc_tiling_on_sc=False),
        scratch_types=[
            pltpu.VMEM((nchunk, CHUNK), jnp.int32),
            pltpu.VMEM((CHUNK, 16), jnp.float32),
            pltpu.VMEM_SHARED((np_, 16), jnp.float32),
        ],
    )
    def deg_kernel(dst_hbm, e0_hbm, zero_hbm, out_hbm, dst_v, e0_v, acc):
        c = lax.axis_index("c")
        s = lax.axis_index("s")
        w = c * NS + s
        pltpu.sync_copy(dst_hbm.at[w], dst_v)
        pltpu.sync_copy(e0_hbm, e0_v)
        pltpu.sync_copy(zero_hbm, acc.at[pl.ds(s * rp, rp)])
        plsc.subcore_barrier()

        @pl.loop(0, nchunk)
        def _(j):
            pltpu.sync_copy(e0_v, acc.at[dst_v.at[j]], add=True)

        plsc.subcore_barrier()
        pltpu.sync_copy(acc.at[pl.ds(s * rp, rp)], out_hbm.at[c, pl.ds(s * rp, rp)])

    return deg_kernel


def _make_prop(np_, rp, nchunk, d):
    mesh = plsc.VectorSubcoreMesh(core_axis_name="c", subcore_axis_name="s")

    @functools.partial(
        pl.kernel,
        out_type=jax.ShapeDtypeStruct((NC, np_, d), jnp.float32),
        mesh=mesh,
        compiler_params=pltpu.CompilerParams(use_tc_tiling_on_sc=False),
        scratch_types=[
            [pltpu.VMEM((1, CHUNK), jnp.int32) for _ in range(4)],
            [pltpu.VMEM((1, CHUNK), jnp.int32) for _ in range(4)],
            [pltpu.VMEM((CHUNK, d), jnp.float32) for _ in range(2)],
            [pltpu.SemaphoreType.DMA for _ in range(4)],
            [pltpu.SemaphoreType.DMA for _ in range(2)],
            [pltpu.SemaphoreType.DMA for _ in range(2)],
            pltpu.VMEM_SHARED((np_, d), jnp.float32),
        ],
    )
    def prop_kernel(g_hbm, src_hbm, dst_hbm, zero_hbm, out_hbm,
                    sidx, didx, bufs, isems, gsems, ssems, acc):
        c = lax.axis_index("c")
        s = lax.axis_index("s")
        w = c * NS + s
        pltpu.sync_copy(zero_hbm, acc.at[pl.ds(s * rp, rp)])
        plsc.subcore_barrier()

        def i_start(j, k):
            pltpu.make_async_copy(src_hbm.at[w, j], sidx[k], isems[k]).start()
            pltpu.make_async_copy(dst_hbm.at[w, j], didx[k], isems[k]).start()

        def i_wait(j, k):
            pltpu.make_async_copy(src_hbm.at[w, j], sidx[k], isems[k]).wait()
            pltpu.make_async_copy(dst_hbm.at[w, j], didx[k], isems[k]).wait()

        def g_start(k, b):
            pltpu.make_async_copy(
                g_hbm.at[sidx[k].at[0]], bufs[b], gsems[b]).start()

        def g_wait(k, b):
            pltpu.make_async_copy(
                g_hbm.at[sidx[k].at[0]], bufs[b], gsems[b]).wait()

        def s_start(k, b):
            pltpu.async_copy(bufs[b], acc.at[didx[k].at[0]], ssems[b], add=True)

        def s_wait(k, b):
            pltpu.make_async_copy(bufs[b], acc.at[didx[k].at[0]], ssems[b]).wait()

        for k in range(3):
            i_start(k, k)
        i_wait(0, 0)
        g_start(0, 0)

        @pl.loop(0, nchunk, step=4)
        def _(g):
            for k in range(4):
                j = g + k
                b = k % 2
                nk = (k + 1) % 4

                g_wait(k, b)
                s_start(k, b)

                @pl.when(j + 1 < nchunk)
                def _():
                    i_wait(j + 1, nk)

                    @pl.when(j >= 1)
                    def _():
                        s_wait((k + 3) % 4, 1 - b)

                    g_start(nk, 1 - b)

                @pl.when(j + 3 < nchunk)
                def _():
                    i_start(j + 3, (k + 3) % 4)

        s_wait((nchunk - 2) % 4, (nchunk - 2) % 2)
        s_wait((nchunk - 1) % 4, (nchunk - 1) % 2)

        plsc.subcore_barrier()
        pltpu.sync_copy(acc.at[pl.ds(s * rp, rp)],
                        out_hbm.at[c, pl.ds(s * rp, rp)])

    return prop_kernel


def _tc_first(np_, r, din, d):
    def body(xr, w1r, dpr, gr, dsr):
        dp = dpr[...]
        deg = dp[0] + dp[1] + 1.0
        dsv = lax.rsqrt(deg)
        h = jnp.dot(xr[...], w1r[...], preferred_element_type=jnp.float32)
        gr[...] = h * dsv[:, None]
        dsr[...] = dsv[:, None]

    return pl.pallas_call(
        body,
        grid=(np_ // r,),
        in_specs=[
            pl.BlockSpec((r, din), lambda i: (i, 0)),
            pl.BlockSpec((din, d), lambda i: (0, 0)),
            pl.BlockSpec((2, r), lambda i: (0, i)),
        ],
        out_specs=[
            pl.BlockSpec((r, d), lambda i: (i, 0)),
            pl.BlockSpec((r, 1), lambda i: (i, 0)),
        ],
        out_shape=[
            jax.ShapeDtypeStruct((np_, d), jnp.float32),
            jax.ShapeDtypeStruct((np_, 1), jnp.float32),
        ],
    )


def _tc_mid(np_, r, d, dout):
    def body(pr, gr, dsr, br, w2r, outr):
        p = pr[...]
        dsv = dsr[...]
        h = jnp.maximum(dsv * (p[0] + p[1] + gr[...]) + br[...], 0.0)
        outr[...] = jnp.dot(h, w2r[...], preferred_element_type=jnp.float32) * dsv

    return pl.pallas_call(
        body,
        grid=(np_ // r,),
        in_specs=[
            pl.BlockSpec((2, r, d), lambda i: (0, i, 0)),
            pl.BlockSpec((r, d), lambda i: (i, 0)),
            pl.BlockSpec((r, 1), lambda i: (i, 0)),
            pl.BlockSpec((1, d), lambda i: (0, 0)),
            pl.BlockSpec((d, dout), lambda i: (0, 0)),
        ],
        out_specs=pl.BlockSpec((r, dout), lambda i: (i, 0)),
        out_shape=jax.ShapeDtypeStruct((np_, dout), jnp.float32),
    )


def _tc_last(np_, r, d):
    def body(pr, gr, dsr, br, wlr, blr, outr):
        p = pr[...]
        dsv = dsr[...]
        h = jnp.maximum(dsv * (p[0] + p[1] + gr[...]) + br[...], 0.0)
        outr[...] = jnp.dot(h, wlr[...], preferred_element_type=jnp.float32) + blr[...]

    return pl.pallas_call(
        body,
        grid=(np_ // r,),
        in_specs=[
            pl.BlockSpec((2, r, d), lambda i: (0, i, 0)),
            pl.BlockSpec((r, d), lambda i: (i, 0)),
            pl.BlockSpec((r, 1), lambda i: (i, 0)),
            pl.BlockSpec((1, d), lambda i: (0, 0)),
            pl.BlockSpec((d, 1), lambda i: (0, 0)),
            pl.BlockSpec((1, 1), lambda i: (0, 0)),
        ],
        out_specs=pl.BlockSpec((r, 1), lambda i: (i, 0)),
        out_shape=jax.ShapeDtypeStruct((np_, 1), jnp.float32),
    )


def kernel(x, edge_index, W1, b1, W2, b2, Wl, bl):
    n, din = x.shape
    e = edge_index.shape[1]
    d = W1.shape[1]
    dout = W2.shape[1]

    rp = _round_up(-(-(n + 1) // NS), 128)
    np_ = rp * NS
    e_pad = _round_up(e, NW * CHUNK * 4)
    nchunk = e_pad // (NW * CHUNK)

    src = edge_index[0].astype(jnp.int32)
    dst = edge_index[1].astype(jnp.int32)
    fill = jnp.full((e_pad - e,), n, jnp.int32)
    src4 = jnp.concatenate([src, fill]).reshape(NW, nchunk, 1, CHUNK)
    dst4 = jnp.concatenate([dst, fill]).reshape(NW, nchunk, 1, CHUNK)
    dst3 = dst4.reshape(NW, nchunk, CHUNK)

    xp = jnp.pad(x, ((0, np_ - n), (0, 0)))
    e0 = jnp.zeros((CHUNK, 16), jnp.float32).at[:, 0].set(1.0)
    z16 = jnp.zeros((rp, 16), jnp.float32)
    zd = jnp.zeros((rp, d), jnp.float32)

    degp = _make_deg(np_, rp, nchunk)(dst3, e0, z16)
    deg2 = degp[:, :, 0]

    g1, ds = _tc_first(np_, rp, din, d)(xp, W1, deg2)
    s1 = _make_prop(np_, rp, nchunk, d)(g1, src4, dst4, zd)
    g2 = _tc_mid(np_, rp, d, dout)(s1, g1, ds, b1.reshape(1, d), W2)
    s2 = _make_prop(np_, rp, nchunk, dout)(g2, src4, dst4, zd)
    out = _tc_last(np_, rp, dout)(s2, g2, ds, b2.reshape(1, dout),
                                  Wl, bl.reshape(1, 1))
    return out[:n]


# speedup vs baseline: 2.0942x; 2.0942x over previous
import functools

import jax
import jax.numpy as jnp
from jax import lax
from jax.experimental import pallas as pl
from jax.experimental.pallas import tpu as pltpu
from jax.experimental.pallas import tpu_sc as plsc

NC = 2
NS = 16
NW = NC * NS
CHUNK = 88


def _round_up(a, m):
    return (a + m - 1) // m * m


def _make_deg(np_, rp, nchunk):
    mesh = plsc.VectorSubcoreMesh(core_axis_name="c", subcore_axis_name="s")

    @functools.partial(
        pl.kernel,
        out_type=jax.ShapeDtypeStruct((NC, np_, 16), jnp.float32),
        mesh=mesh,
        compiler_params=pltpu.CompilerParams(use_tc_tiling_on_sc=False),
        scratch_types=[
            pltpu.VMEM((nchunk, CHUNK), jnp.int32),
            pltpu.VMEM((CHUNK, 16), jnp.float32),
            pltpu.VMEM_SHARED((np_, 16), jnp.float32),
        ],
    )
    def deg_kernel(dst_hbm, e0_hbm, zero_hbm, out_hbm, dst_v, e0_v, acc):
        c = lax.axis_index("c")
        s = lax.axis_index("s")
        w = c * NS + s
        pltpu.sync_copy(dst_hbm.at[w], dst_v)
        pltpu.sync_copy(e0_hbm, e0_v)
        pltpu.sync_copy(zero_hbm, acc.at[pl.ds(s * rp, rp)])
        plsc.subcore_barrier()

        @pl.loop(0, nchunk)
        def _(j):
            pltpu.sync_copy(e0_v, acc.at[dst_v.at[j]], add=True)

        plsc.subcore_barrier()
        pltpu.sync_copy(acc.at[pl.ds(s * rp, rp)], out_hbm.at[c, pl.ds(s * rp, rp)])

    return deg_kernel


def _make_prop(np_, rp, nchunk, d):
    mesh = plsc.VectorSubcoreMesh(core_axis_name="c", subcore_axis_name="s")

    @functools.partial(
        pl.kernel,
        out_type=jax.ShapeDtypeStruct((NC, np_, d), jnp.float32),
        mesh=mesh,
        compiler_params=pltpu.CompilerParams(use_tc_tiling_on_sc=False),
        scratch_types=[
            pltpu.VMEM((nchunk, CHUNK), jnp.int32),
            pltpu.VMEM((nchunk, CHUNK), jnp.int32),
            [pltpu.VMEM((CHUNK, d), jnp.float32) for _ in range(2)],
            [pltpu.SemaphoreType.DMA for _ in range(2)],
            [pltpu.SemaphoreType.DMA for _ in range(2)],
            pltpu.VMEM_SHARED((np_, d), jnp.float32),
        ],
    )
    def prop_kernel(g_hbm, src_hbm, dst_hbm, zero_hbm, out_hbm,
                    src_v, dst_v, bufs, gsems, ssems, acc):
        c = lax.axis_index("c")
        s = lax.axis_index("s")
        w = c * NS + s
        pltpu.sync_copy(src_hbm.at[w], src_v)
        pltpu.sync_copy(dst_hbm.at[w], dst_v)
        pltpu.sync_copy(zero_hbm, acc.at[pl.ds(s * rp, rp)])
        plsc.subcore_barrier()

        def g_start(j, b):
            pltpu.make_async_copy(
                g_hbm.at[src_v.at[j]], bufs[b], gsems[b]).start()

        def g_wait(j, b):
            pltpu.make_async_copy(
                g_hbm.at[src_v.at[j]], bufs[b], gsems[b]).wait()

        def s_start(j, b):
            pltpu.async_copy(bufs[b], acc.at[dst_v.at[j]], ssems[b], add=True)

        def s_wait(j, b):
            pltpu.make_async_copy(bufs[b], acc.at[dst_v.at[j]], ssems[b]).wait()

        g_start(0, 0)

        @pl.loop(0, nchunk, step=2)
        def _(g):
            for b in range(2):
                j = g + b
                g_wait(j, b)
                s_start(j, b)

                @pl.when(j + 1 < nchunk)
                def _():
                    @pl.when(j >= 1)
                    def _():
                        s_wait(j - 1, 1 - b)

                    g_start(j + 1, 1 - b)

        s_wait(nchunk - 2, nchunk % 2)
        s_wait(nchunk - 1, 1 - nchunk % 2)

        plsc.subcore_barrier()
        pltpu.sync_copy(acc.at[pl.ds(s * rp, rp)],
                        out_hbm.at[c, pl.ds(s * rp, rp)])

    return prop_kernel


def _tc_first(np_, r, din, d):
    def body(xr, w1r, dpr, gr, dsr):
        dp = dpr[...]
        deg = dp[0] + dp[1] + 1.0
        dsv = lax.rsqrt(deg)
        h = jnp.dot(xr[...], w1r[...], preferred_element_type=jnp.float32)
        gr[...] = h * dsv[:, None]
        dsr[...] = dsv[:, None]

    return pl.pallas_call(
        body,
        grid=(np_ // r,),
        in_specs=[
            pl.BlockSpec((r, din), lambda i: (i, 0)),
            pl.BlockSpec((din, d), lambda i: (0, 0)),
            pl.BlockSpec((2, r), lambda i: (0, i)),
        ],
        out_specs=[
            pl.BlockSpec((r, d), lambda i: (i, 0)),
            pl.BlockSpec((r, 1), lambda i: (i, 0)),
        ],
        out_shape=[
            jax.ShapeDtypeStruct((np_, d), jnp.float32),
            jax.ShapeDtypeStruct((np_, 1), jnp.float32),
        ],
    )


def _tc_mid(np_, r, d, dout):
    def body(pr, gr, dsr, br, w2r, outr):
        p = pr[...]
        dsv = dsr[...]
        h = jnp.maximum(dsv * (p[0] + p[1] + gr[...]) + br[...], 0.0)
        outr[...] = jnp.dot(h, w2r[...], preferred_element_type=jnp.float32) * dsv

    return pl.pallas_call(
        body,
        grid=(np_ // r,),
        in_specs=[
            pl.BlockSpec((2, r, d), lambda i: (0, i, 0)),
            pl.BlockSpec((r, d), lambda i: (i, 0)),
            pl.BlockSpec((r, 1), lambda i: (i, 0)),
            pl.BlockSpec((1, d), lambda i: (0, 0)),
            pl.BlockSpec((d, dout), lambda i: (0, 0)),
        ],
        out_specs=pl.BlockSpec((r, dout), lambda i: (i, 0)),
        out_shape=jax.ShapeDtypeStruct((np_, dout), jnp.float32),
    )


def _tc_last(np_, r, d):
    def body(pr, gr, dsr, br, wlr, blr, outr):
        p = pr[...]
        dsv = dsr[...]
        h = jnp.maximum(dsv * (p[0] + p[1] + gr[...]) + br[...], 0.0)
        outr[...] = jnp.dot(h, wlr[...], preferred_element_type=jnp.float32) + blr[...]

    return pl.pallas_call(
        body,
        grid=(np_ // r,),
        in_specs=[
            pl.BlockSpec((2, r, d), lambda i: (0, i, 0)),
            pl.BlockSpec((r, d), lambda i: (i, 0)),
            pl.BlockSpec((r, 1), lambda i: (i, 0)),
            pl.BlockSpec((1, d), lambda i: (0, 0)),
            pl.BlockSpec((d, 1), lambda i: (0, 0)),
            pl.BlockSpec((1, 1), lambda i: (0, 0)),
        ],
        out_specs=pl.BlockSpec((r, 1), lambda i: (i, 0)),
        out_shape=jax.ShapeDtypeStruct((np_, 1), jnp.float32),
    )


def kernel(x, edge_index, W1, b1, W2, b2, Wl, bl):
    n, din = x.shape
    e = edge_index.shape[1]
    d = W1.shape[1]
    dout = W2.shape[1]

    rp = _round_up(-(-(n + 1) // NS), 128)
    np_ = rp * NS
    e_pad = _round_up(e, NW * CHUNK * 2)
    nchunk = e_pad // (NW * CHUNK)

    src = edge_index[0].astype(jnp.int32)
    dst = edge_index[1].astype(jnp.int32)
    fill = jnp.full((e_pad - e,), n, jnp.int32)
    src3 = jnp.concatenate([src, fill]).reshape(NW, nchunk, CHUNK)
    dst3 = jnp.concatenate([dst, fill]).reshape(NW, nchunk, CHUNK)

    xp = jnp.pad(x, ((0, np_ - n), (0, 0)))
    e0 = jnp.zeros((CHUNK, 16), jnp.float32).at[:, 0].set(1.0)
    z16 = jnp.zeros((rp, 16), jnp.float32)
    zd = jnp.zeros((rp, d), jnp.float32)

    degp = _make_deg(np_, rp, nchunk)(dst3, e0, z16)
    deg2 = degp[:, :, 0]

    g1, ds = _tc_first(np_, rp, din, d)(xp, W1, deg2)
    s1 = _make_prop(np_, rp, nchunk, d)(g1, src3, dst3, zd)
    g2 = _tc_mid(np_, rp, d, dout)(s1, g1, ds, b1.reshape(1, d), W2)
    s2 = _make_prop(np_, rp, nchunk, dout)(g2, src3, dst3, zd)
    out = _tc_last(np_, rp, dout)(s2, g2, ds, b2.reshape(1, dout),
                                  Wl, bl.reshape(1, 1))
    return out[:n]


# speedup vs baseline: 2.1235x; 1.0140x over previous
import functools

import jax
import jax.numpy as jnp
from jax import lax
from jax.experimental import pallas as pl
from jax.experimental.pallas import tpu as pltpu
from jax.experimental.pallas import tpu_sc as plsc

NC = 2
NS = 16
NW = NC * NS
CHUNK = 88


def _round_up(a, m):
    return (a + m - 1) // m * m


def _make_deg(np_, rp, nchunk):
    mesh = plsc.VectorSubcoreMesh(core_axis_name="c", subcore_axis_name="s")

    @functools.partial(
        pl.kernel,
        out_type=jax.ShapeDtypeStruct((NC, np_, 16), jnp.float32),
        mesh=mesh,
        compiler_params=pltpu.CompilerParams(use_tc_tiling_on_sc=False),
        scratch_types=[
            pltpu.VMEM((nchunk, CHUNK), jnp.int32),
            pltpu.VMEM((CHUNK, 16), jnp.float32),
            pltpu.SemaphoreType.DMA,
            pltpu.VMEM_SHARED((np_, 16), jnp.float32),
        ],
    )
    def deg_kernel(dst_hbm, e0_hbm, zero_hbm, out_hbm, dst_v, e0_v, sem, acc):
        c = lax.axis_index("c")
        s = lax.axis_index("s")
        w = c * NS + s
        pltpu.sync_copy(dst_hbm.at[w], dst_v)
        pltpu.sync_copy(e0_hbm, e0_v)
        pltpu.sync_copy(zero_hbm, acc.at[pl.ds(s * rp, rp)])
        plsc.subcore_barrier()

        pltpu.async_copy(e0_v, acc.at[dst_v.at[0]], sem, add=True)

        @pl.loop(1, nchunk)
        def _(j):
            pltpu.async_copy(e0_v, acc.at[dst_v.at[j]], sem, add=True)
            pltpu.make_async_copy(e0_v, acc.at[dst_v.at[0]], sem).wait()

        pltpu.make_async_copy(e0_v, acc.at[dst_v.at[0]], sem).wait()

        plsc.subcore_barrier()
        pltpu.sync_copy(acc.at[pl.ds(s * rp, rp)], out_hbm.at[c, pl.ds(s * rp, rp)])

    return deg_kernel


def _make_prop(np_, rp, nchunk, d):
    mesh = plsc.VectorSubcoreMesh(core_axis_name="c", subcore_axis_name="s")

    @functools.partial(
        pl.kernel,
        out_type=jax.ShapeDtypeStruct((NC, np_, d), jnp.float32),
        mesh=mesh,
        compiler_params=pltpu.CompilerParams(use_tc_tiling_on_sc=False),
        scratch_types=[
            pltpu.VMEM((nchunk, CHUNK), jnp.int32),
            pltpu.VMEM((nchunk, CHUNK), jnp.int32),
            [pltpu.VMEM((CHUNK, d), jnp.float32) for _ in range(2)],
            [pltpu.SemaphoreType.DMA for _ in range(2)],
            [pltpu.SemaphoreType.DMA for _ in range(2)],
            pltpu.VMEM_SHARED((np_, d), jnp.float32),
        ],
    )
    def prop_kernel(g_hbm, src_hbm, dst_hbm, zero_hbm, out_hbm,
                    src_v, dst_v, bufs, gsems, ssems, acc):
        c = lax.axis_index("c")
        s = lax.axis_index("s")
        w = c * NS + s
        pltpu.sync_copy(src_hbm.at[w], src_v)
        pltpu.sync_copy(dst_hbm.at[w], dst_v)
        pltpu.sync_copy(zero_hbm, acc.at[pl.ds(s * rp, rp)])
        plsc.subcore_barrier()

        def g_start(j, b):
            pltpu.make_async_copy(
                g_hbm.at[src_v.at[j]], bufs[b], gsems[b]).start()

        def g_wait(j, b):
            pltpu.make_async_copy(
                g_hbm.at[src_v.at[j]], bufs[b], gsems[b]).wait()

        def s_start(j, b):
            pltpu.async_copy(bufs[b], acc.at[dst_v.at[j]], ssems[b], add=True)

        def s_wait(j, b):
            pltpu.make_async_copy(bufs[b], acc.at[dst_v.at[j]], ssems[b]).wait()

        g_start(0, 0)

        @pl.loop(0, nchunk, step=2)
        def _(g):
            for b in range(2):
                j = g + b
                g_wait(j, b)
                s_start(j, b)

                @pl.when(j + 1 < nchunk)
                def _():
                    @pl.when(j >= 1)
                    def _():
                        s_wait(j - 1, 1 - b)

                    g_start(j + 1, 1 - b)

        s_wait(nchunk - 2, nchunk % 2)
        s_wait(nchunk - 1, 1 - nchunk % 2)

        plsc.subcore_barrier()
        pltpu.sync_copy(acc.at[pl.ds(s * rp, rp)],
                        out_hbm.at[c, pl.ds(s * rp, rp)])

    return prop_kernel


def _tc_first(np_, r, din, d):
    def body(xr, w1r, dpr, gr, dsr):
        dp = dpr[...]
        deg = dp[0, :, 0] + dp[1, :, 0] + 1.0
        dsv = lax.rsqrt(deg)
        h = jnp.dot(xr[...], w1r[...], preferred_element_type=jnp.float32)
        gr[...] = h * dsv[:, None]
        dsr[...] = dsv[:, None]

    return pl.pallas_call(
        body,
        grid=(np_ // r,),
        in_specs=[
            pl.BlockSpec((r, din), lambda i: (i, 0)),
            pl.BlockSpec((din, d), lambda i: (0, 0)),
            pl.BlockSpec((2, r, 16), lambda i: (0, i, 0)),
        ],
        out_specs=[
            pl.BlockSpec((r, d), lambda i: (i, 0)),
            pl.BlockSpec((r, 1), lambda i: (i, 0)),
        ],
        out_shape=[
            jax.ShapeDtypeStruct((np_, d), jnp.float32),
            jax.ShapeDtypeStruct((np_, 1), jnp.float32),
        ],
    )


def _tc_mid(np_, r, d, dout):
    def body(pr, gr, dsr, br, w2r, outr):
        p = pr[...]
        dsv = dsr[...]
        h = jnp.maximum(dsv * (p[0] + p[1] + gr[...]) + br[...], 0.0)
        outr[...] = jnp.dot(h, w2r[...], preferred_element_type=jnp.float32) * dsv

    return pl.pallas_call(
        body,
        grid=(np_ // r,),
        in_specs=[
            pl.BlockSpec((2, r, d), lambda i: (0, i, 0)),
            pl.BlockSpec((r, d), lambda i: (i, 0)),
            pl.BlockSpec((r, 1), lambda i: (i, 0)),
            pl.BlockSpec((1, d), lambda i: (0, 0)),
            pl.BlockSpec((d, dout), lambda i: (0, 0)),
        ],
        out_specs=pl.BlockSpec((r, dout), lambda i: (i, 0)),
        out_shape=jax.ShapeDtypeStruct((np_, dout), jnp.float32),
    )


def _tc_last(np_, r, d):
    def body(pr, gr, dsr, br, wlr, blr, outr):
        p = pr[...]
        dsv = dsr[...]
        h = jnp.maximum(dsv * (p[0] + p[1] + gr[...]) + br[...], 0.0)
        outr[...] = jnp.dot(h, wlr[...], preferred_element_type=jnp.float32) + blr[...]

    return pl.pallas_call(
        body,
        grid=(np_ // r,),
        in_specs=[
            pl.BlockSpec((2, r, d), lambda i: (0, i, 0)),
            pl.BlockSpec((r, d), lambda i: (i, 0)),
            pl.BlockSpec((r, 1), lambda i: (i, 0)),
            pl.BlockSpec((1, d), lambda i: (0, 0)),
            pl.BlockSpec((d, 1), lambda i: (0, 0)),
            pl.BlockSpec((1, 1), lambda i: (0, 0)),
        ],
        out_specs=pl.BlockSpec((r, 1), lambda i: (i, 0)),
        out_shape=jax.ShapeDtypeStruct((np_, 1), jnp.float32),
    )


def kernel(x, edge_index, W1, b1, W2, b2, Wl, bl):
    n, din = x.shape
    e = edge_index.shape[1]
    d = W1.shape[1]
    dout = W2.shape[1]

    rp = _round_up(-(-(n + 1) // NS), 128)
    np_ = rp * NS
    e_pad = _round_up(e, NW * CHUNK * 2)
    nchunk = e_pad // (NW * CHUNK)

    src = edge_index[0].astype(jnp.int32)
    dst = edge_index[1].astype(jnp.int32)
    fill = jnp.full((e_pad - e,), n, jnp.int32)
    src3 = jnp.concatenate([src, fill]).reshape(NW, nchunk, CHUNK)
    dst3 = jnp.concatenate([dst, fill]).reshape(NW, nchunk, CHUNK)

    e0 = jnp.zeros((CHUNK, 16), jnp.float32).at[:, 0].set(1.0)
    z16 = jnp.zeros((rp, 16), jnp.float32)
    zd = jnp.zeros((rp, d), jnp.float32)

    degp = _make_deg(np_, rp, nchunk)(dst3, e0, z16)

    g1, ds = _tc_first(np_, rp, din, d)(x, W1, degp)
    s1 = _make_prop(np_, rp, nchunk, d)(g1, src3, dst3, zd)
    g2 = _tc_mid(np_, rp, d, dout)(s1, g1, ds, b1.reshape(1, d), W2)
    s2 = _make_prop(np_, rp, nchunk, dout)(g2, src3, dst3, zd)
    out = _tc_last(np_, rp, dout)(s2, g2, ds, b2.reshape(1, dout),
                                  Wl, bl.reshape(1, 1))
    return out[:n]
